# baseline (speedup 1.0000x reference)
import functools

import jax
import jax.numpy as jnp
from jax import lax
from jax.experimental import pallas as pl
from jax.experimental.pallas import tpu as pltpu
from jax.experimental.pallas import tpu_sc as plsc

_N = 10000
_E = 320000
_C = 128
_H = 64
_DE = 16
_L = 3

_TILES = 16
_K = 64
_EPT = 20480
_EP = _EPT * _TILES
_NP = 10112
_RPT = _NP // _TILES


def _edge_proj(ea_p, We, be2, lew, leb):
    Eb = 2048

    def body(ea_ref, we_ref, be_ref, lw_ref, lb_ref, o0, o1, o2):
        t = jnp.dot(ea_ref[...], we_ref[...],
                    preferred_element_type=jnp.float32) + be_ref[...]
        t = t * jax.nn.sigmoid(t)
        outs = (o0, o1, o2)
        for l in range(_L):
            r = jnp.dot(t, lw_ref[l],
                        preferred_element_type=jnp.float32) + lb_ref[l]
            outs[l][0] = r[:, :_H]
            outs[l][1] = r[:, _H:]

    return pl.pallas_call(
        body,
        grid=(_EP // Eb,),
        in_specs=[
            pl.BlockSpec((Eb, _DE), lambda i: (i, 0)),
            pl.BlockSpec((_DE, _C), lambda i: (0, 0)),
            pl.BlockSpec((1, _C), lambda i: (0, 0)),
            pl.BlockSpec((_L, _C, _C), lambda i: (0, 0, 0)),
            pl.BlockSpec((_L, 1, _C), lambda i: (0, 0, 0)),
        ],
        out_specs=[pl.BlockSpec((2, Eb, _H), lambda i: (0, i, 0))] * _L,
        out_shape=[jax.ShapeDtypeStruct((2, _EP, _H), jnp.float32)] * _L,
    )(ea_p, We, be2, lew, leb)


def _sc_layer(ep, src, dst, x):
    mesh = plsc.VectorSubcoreMesh(core_axis_name="c", subcore_axis_name="s",
                                  num_cores=2, num_subcores=_TILES)

    @functools.partial(
        pl.kernel,
        out_type=jax.ShapeDtypeStruct((2, _NP, _H), jnp.float32),
        mesh=mesh,
        scratch_types=[
            pltpu.VMEM_SHARED((_NP, _H), jnp.float32),
            pltpu.VMEM_SHARED((_NP, _H), jnp.float32),
            pltpu.VMEM((2, _K, _H), jnp.float32),
            pltpu.VMEM((2, _K, _H), jnp.float32),
            pltpu.VMEM((4, _K), jnp.int32),
            pltpu.VMEM((4, _K), jnp.int32),
            pltpu.SemaphoreType.DMA((2,)),
            pltpu.SemaphoreType.DMA((4,)),
            pltpu.SemaphoreType.DMA((2,)),
            pltpu.SemaphoreType.DMA((2,)),
        ],
    )
    def k(ep_h, src_h, dst_h, x_h, out_h, xsh, agg, epb, xgb, srcb, dstb,
          sem_ep, sem_ix, sem_g, sem_sc):
        c = lax.axis_index("c")
        s = lax.axis_index("s")
        r0 = s * _RPT
        nck = _EPT // _K
        e_base = s * _EPT

        pltpu.sync_copy(x_h.at[c, pl.ds(r0, _RPT)], xsh.at[pl.ds(r0, _RPT)])
        pltpu.sync_copy(x_h.at[c, pl.ds(r0, _RPT)], agg.at[pl.ds(r0, _RPT)])
        plsc.subcore_barrier()

        def fetch(g, b2, b4):
            e0 = e_base + g * _K
            pltpu.async_copy(src_h.at[pl.ds(e0, _K)], srcb.at[b4],
                             sem_ix.at[b4])
            pltpu.async_copy(dst_h.at[pl.ds(e0, _K)], dstb.at[b4],
                             sem_ix.at[b4])
            pltpu.async_copy(ep_h.at[c, pl.ds(e0, _K)], epb.at[b2],
                             sem_ep.at[b2])

        def wait_idx(b4):
            pltpu.make_async_copy(src_h.at[pl.ds(0, _K)], srcb.at[b4],
                                  sem_ix.at[b4]).wait()
            pltpu.make_async_copy(dst_h.at[pl.ds(0, _K)], dstb.at[b4],
                                  sem_ix.at[b4]).wait()

        def gather(b2, b4):
            pltpu.async_copy(xsh.at[srcb.at[b4]], xgb.at[b2], sem_g.at[b2])

        def wait_gather(b2, b4):
            pltpu.make_async_copy(xsh.at[srcb.at[b4]], xgb.at[b2],
                                  sem_g.at[b2]).wait()

        def wait_ep(b2):
            pltpu.make_async_copy(ep_h.at[0, pl.ds(0, _K)], epb.at[b2],
                                  sem_ep.at[b2]).wait()

        def scatter(b2, b4):
            pltpu.async_copy(xgb.at[b2], agg.at[dstb.at[b4]], sem_sc.at[b2],
                             add=True)

        def wait_scatter(b2, b4):
            pltpu.make_async_copy(xgb.at[b2], agg.at[dstb.at[b4]],
                                  sem_sc.at[b2]).wait()

        def compute(b2):
            @pl.loop(0, _K, unroll=2)
            def _row(r):
                for cb in range(_H // 16):
                    sl = pl.ds(cb * 16, 16)
                    xgb[b2, r, sl] = jnp.maximum(
                        xgb[b2, r, sl] + epb[b2, r, sl], 0.0)

        fetch(0, 0, 0)
        wait_idx(0)
        gather(0, 0)
        fetch(1, 1, 1)

        @pl.loop(0, nck // 4)
        def _outer(g4):
            for u in range(4):
                b2 = u % 2
                nb2 = (u + 1) % 2
                g = g4 * 4 + u

                @pl.when(g + 1 < nck)
                def _():
                    wait_idx((u + 1) % 4)

                    @pl.when(g >= 1)
                    def _():
                        wait_scatter(nb2, (u + 3) % 4)

                    gather(nb2, (u + 1) % 4)

                wait_ep(b2)
                wait_gather(b2, u % 4)
                compute(b2)
                scatter(b2, u % 4)

                @pl.when(g + 2 < nck)
                def _():
                    fetch(g + 2, b2, (u + 2) % 4)

        wait_scatter(0, (nck - 2) % 4)
        wait_scatter(1, (nck - 1) % 4)
        plsc.subcore_barrier()
        pltpu.sync_copy(agg.at[pl.ds(r0, _RPT)], out_h.at[c, pl.ds(r0, _RPT)])

    return k(ep, src, dst, x)


def _node_mlp(hp, x, w1l, b1l, w2l, b2l, gl, btl):
    Nb = _RPT

    def body(hp_ref, x_ref, w1_ref, b1_ref, w2_ref, b2_ref, g_ref, bt_ref, o_ref):
        xv = jnp.concatenate([x_ref[0], x_ref[1]], axis=-1)
        hp_v = jnp.concatenate([hp_ref[0], hp_ref[1]], axis=-1)
        t = jnp.dot(hp_v, w1_ref[...],
                    preferred_element_type=jnp.float32) + b1_ref[...]
        t = t * jax.nn.sigmoid(t)
        h = jnp.dot(t, w2_ref[...],
                    preferred_element_type=jnp.float32) + b2_ref[...]
        y = xv + h
        mu = jnp.mean(y, axis=-1, keepdims=True)
        d = y - mu
        var = jnp.mean(d * d, axis=-1, keepdims=True)
        o = d * lax.rsqrt(var + 1e-5) * g_ref[...] + bt_ref[...]
        o_ref[0] = o[:, :_H]
        o_ref[1] = o[:, _H:]

    full = lambda i: (0, 0)
    return pl.pallas_call(
        body,
        grid=(_NP // Nb,),
        in_specs=[
            pl.BlockSpec((2, Nb, _H), lambda i: (0, i, 0)),
            pl.BlockSpec((2, Nb, _H), lambda i: (0, i, 0)),
            pl.BlockSpec((_C, _C), full),
            pl.BlockSpec((1, _C), full),
            pl.BlockSpec((_C, _C), full),
            pl.BlockSpec((1, _C), full),
            pl.BlockSpec((1, _C), full),
            pl.BlockSpec((1, _C), full),
        ],
        out_specs=pl.BlockSpec((2, Nb, _H), lambda i: (0, i, 0)),
        out_shape=jax.ShapeDtypeStruct((2, _NP, _H), jnp.float32),
    )(hp, x, w1l, b1l, w2l, b2l, gl, btl)


def kernel(x, edge_index, edge_attr, We, be, lin_e_w, lin_e_b, w1, b1, w2, b2,
           ln_g, ln_b):
    pad = _EP - _E
    src = jnp.pad(edge_index[0], (0, pad))
    pad_dst = _N + (jnp.arange(pad, dtype=jnp.int32) % (_NP - _N))
    dst = jnp.concatenate([edge_index[1], pad_dst])
    ea_p = jnp.pad(edge_attr, ((0, pad), (0, 0)))
    eps = _edge_proj(ea_p, We, be.reshape(1, _C), lin_e_w,
                     lin_e_b.reshape(_L, 1, _C))
    xp = jnp.pad(x, ((0, _NP - _N), (0, 0)))
    x2 = jnp.stack([xp[:, :_H], xp[:, _H:]])
    for l in range(_L):
        hp2 = _sc_layer(eps[l], src, dst, x2)
        x2 = _node_mlp(hp2, x2, w1[l], b1[l].reshape(1, _C), w2[l],
                       b2[l].reshape(1, _C), ln_g[l].reshape(1, _C),
                       ln_b[l].reshape(1, _C))
    return jnp.concatenate([x2[0, :_N], x2[1, :_N]], axis=1)


# speedup vs baseline: 1.1565x; 1.1565x over previous
import functools

import jax
import jax.numpy as jnp
from jax import lax
from jax.experimental import pallas as pl
from jax.experimental.pallas import tpu as pltpu
from jax.experimental.pallas import tpu_sc as plsc

_N = 10000
_E = 320000
_C = 128
_H = 64
_DE = 16
_L = 3

_TILES = 16
_K = 64
_EPT = 20480
_EP = _EPT * _TILES
_NP = 10112
_RPT = _NP // _TILES


def _edge_proj(ea_p, We, be2, lew, leb):
    Eb = 1024
    nblk = _EP // 2 // Eb

    def body(ea_a, ea_b, we_ref, be_ref, lw_ref, lb_ref, o0, o1, o2):
        ta = jnp.dot(ea_a[...], we_ref[...],
                     preferred_element_type=jnp.float32) + be_ref[...]
        ta = ta * jax.nn.sigmoid(ta)
        tb = jnp.dot(ea_b[...], we_ref[...],
                     preferred_element_type=jnp.float32) + be_ref[...]
        tb = tb * jax.nn.sigmoid(tb)
        outs = (o0, o1, o2)
        for l in range(_L):
            ra = jnp.dot(ta, lw_ref[l],
                         preferred_element_type=jnp.float32) + lb_ref[l]
            rb = jnp.dot(tb, lw_ref[l],
                         preferred_element_type=jnp.float32) + lb_ref[l]
            outs[l][0] = jnp.concatenate([ra[:, :_H], rb[:, :_H]], axis=1)
            outs[l][1] = jnp.concatenate([ra[:, _H:], rb[:, _H:]], axis=1)

    return pl.pallas_call(
        body,
        grid=(nblk,),
        in_specs=[
            pl.BlockSpec((Eb, _DE), lambda i: (i, 0)),
            pl.BlockSpec((Eb, _DE), lambda i: (i + nblk, 0)),
            pl.BlockSpec((_DE, _C), lambda i: (0, 0)),
            pl.BlockSpec((1, _C), lambda i: (0, 0)),
            pl.BlockSpec((_L, _C, _C), lambda i: (0, 0, 0)),
            pl.BlockSpec((_L, 1, _C), lambda i: (0, 0, 0)),
        ],
        out_specs=[pl.BlockSpec((2, Eb, _C), lambda i: (0, i, 0))] * _L,
        out_shape=[jax.ShapeDtypeStruct((2, _EP // 2, _C), jnp.float32)] * _L,
    )(ea_p, ea_p, We, be2, lew, leb)


def _sc_layer(ep, src, dst, x):
    mesh = plsc.VectorSubcoreMesh(core_axis_name="c", subcore_axis_name="s",
                                  num_cores=2, num_subcores=_TILES)

    @functools.partial(
        pl.kernel,
        out_type=jax.ShapeDtypeStruct((2, _NP, _H), jnp.float32),
        mesh=mesh,
        scratch_types=[
            pltpu.VMEM_SHARED((_NP, _H), jnp.float32),
            pltpu.VMEM_SHARED((_NP, _H), jnp.float32),
            pltpu.VMEM((2, _K // 2, _C), jnp.float32),
            pltpu.VMEM((4, _K, _H), jnp.float32),
            pltpu.VMEM((8, _K), jnp.int32),
            pltpu.VMEM((8, _K), jnp.int32),
            pltpu.SemaphoreType.DMA((2,)),
            pltpu.SemaphoreType.DMA((8,)),
            pltpu.SemaphoreType.DMA((4,)),
            pltpu.SemaphoreType.DMA((4,)),
        ],
    )
    def k(ep_h, src_h, dst_h, x_h, out_h, xsh, agg, epb, xgb, srcb, dstb,
          sem_ep, sem_ix, sem_g, sem_sc):
        c = lax.axis_index("c")
        s = lax.axis_index("s")
        r0 = s * _RPT
        nck = _EPT // _K
        e_base = s * _EPT

        pltpu.sync_copy(x_h.at[c, pl.ds(r0, _RPT)], xsh.at[pl.ds(r0, _RPT)])
        pltpu.sync_copy(x_h.at[c, pl.ds(r0, _RPT)], agg.at[pl.ds(r0, _RPT)])
        plsc.subcore_barrier()

        def fetch_idx(g, b8):
            e0 = e_base + g * _K
            pltpu.async_copy(src_h.at[pl.ds(e0, _K)], srcb.at[b8],
                             sem_ix.at[b8])
            pltpu.async_copy(dst_h.at[pl.ds(e0, _K)], dstb.at[b8],
                             sem_ix.at[b8])

        def fetch_ep(g, b2):
            p0 = s * (_EPT // 2) + g * (_K // 2)
            pltpu.async_copy(ep_h.at[c, pl.ds(p0, _K // 2)], epb.at[b2],
                             sem_ep.at[b2])

        def wait_idx(b8):
            pltpu.make_async_copy(src_h.at[pl.ds(0, _K)], srcb.at[b8],
                                  sem_ix.at[b8]).wait()
            pltpu.make_async_copy(dst_h.at[pl.ds(0, _K)], dstb.at[b8],
                                  sem_ix.at[b8]).wait()

        def gather(b4, b8):
            pltpu.async_copy(xsh.at[srcb.at[b8]], xgb.at[b4], sem_g.at[b4])

        def wait_gather(b4, b8):
            pltpu.make_async_copy(xsh.at[srcb.at[b8]], xgb.at[b4],
                                  sem_g.at[b4]).wait()

        def wait_ep(b2):
            pltpu.make_async_copy(ep_h.at[0, pl.ds(0, _K // 2)], epb.at[b2],
                                  sem_ep.at[b2]).wait()

        def scatter(b4, b8):
            pltpu.async_copy(xgb.at[b4], agg.at[dstb.at[b8]], sem_sc.at[b4],
                             add=True)

        def wait_scatter(b4, b8):
            pltpu.make_async_copy(xgb.at[b4], agg.at[dstb.at[b8]],
                                  sem_sc.at[b4]).wait()

        def compute(b4, b2):
            @pl.loop(0, _K // 2, unroll=2)
            def _row(r2):
                for h in range(2):
                    for cb in range(_H // 16):
                        sl = pl.ds(cb * 16, 16)
                        esl = pl.ds(h * _H + cb * 16, 16)
                        xgb[b4, 2 * r2 + h, sl] = jnp.maximum(
                            xgb[b4, 2 * r2 + h, sl] + epb[b2, r2, esl], 0.0)

        for g in range(4):
            fetch_idx(g, g % 8)
        fetch_ep(0, 0)
        fetch_ep(1, 1)
        wait_idx(0)
        gather(0, 0)
        wait_idx(1)
        gather(1, 1)

        @pl.loop(0, nck // 8)
        def _outer(g8):
            for u in range(8):
                b4 = u % 4
                b2 = u % 2
                g = g8 * 8 + u

                @pl.when(g + 2 < nck)
                def _():
                    wait_idx((u + 2) % 8)

                    @pl.when(g >= 2)
                    def _():
                        wait_scatter((u + 2) % 4, (u + 2) % 8)

                    gather((u + 2) % 4, (u + 2) % 8)

                wait_ep(b2)
                wait_gather(b4, u % 8)
                compute(b4, b2)
                scatter(b4, u % 8)

                @pl.when(g + 2 < nck)
                def _():
                    fetch_ep(g + 2, b2)

                @pl.when(g + 4 < nck)
                def _():
                    fetch_idx(g + 4, (u + 4) % 8)

        for g in range(nck - 4, nck):
            wait_scatter(g % 4, g % 8)
        plsc.subcore_barrier()
        pltpu.sync_copy(agg.at[pl.ds(r0, _RPT)], out_h.at[c, pl.ds(r0, _RPT)])

    return k(ep, src, dst, x)


def _node_mlp(hp, x, w1l, b1l, w2l, b2l, gl, btl):
    Nb = _RPT

    def body(hp_ref, x_ref, w1_ref, b1_ref, w2_ref, b2_ref, g_ref, bt_ref, o_ref):
        xv = jnp.concatenate([x_ref[0], x_ref[1]], axis=-1)
        hp_v = jnp.concatenate([hp_ref[0], hp_ref[1]], axis=-1)
        t = jnp.dot(hp_v, w1_ref[...],
                    preferred_element_type=jnp.float32) + b1_ref[...]
        t = t * jax.nn.sigmoid(t)
        h = jnp.dot(t, w2_ref[...],
                    preferred_element_type=jnp.float32) + b2_ref[...]
        y = xv + h
        mu = jnp.mean(y, axis=-1, keepdims=True)
        d = y - mu
        var = jnp.mean(d * d, axis=-1, keepdims=True)
        o = d * lax.rsqrt(var + 1e-5) * g_ref[...] + bt_ref[...]
        o_ref[0] = o[:, :_H]
        o_ref[1] = o[:, _H:]

    full = lambda i: (0, 0)
    return pl.pallas_call(
        body,
        grid=(_NP // Nb,),
        in_specs=[
            pl.BlockSpec((2, Nb, _H), lambda i: (0, i, 0)),
            pl.BlockSpec((2, Nb, _H), lambda i: (0, i, 0)),
            pl.BlockSpec((_C, _C), full),
            pl.BlockSpec((1, _C), full),
            pl.BlockSpec((_C, _C), full),
            pl.BlockSpec((1, _C), full),
            pl.BlockSpec((1, _C), full),
            pl.BlockSpec((1, _C), full),
        ],
        out_specs=pl.BlockSpec((2, Nb, _H), lambda i: (0, i, 0)),
        out_shape=jax.ShapeDtypeStruct((2, _NP, _H), jnp.float32),
    )(hp, x, w1l, b1l, w2l, b2l, gl, btl)


def kernel(x, edge_index, edge_attr, We, be, lin_e_w, lin_e_b, w1, b1, w2, b2,
           ln_g, ln_b):
    pad = _EP - _E
    src = jnp.pad(edge_index[0], (0, pad))
    pad_dst = _N + (jnp.arange(pad, dtype=jnp.int32) % (_NP - _N))
    dst = jnp.concatenate([edge_index[1], pad_dst])
    half = _EP // 2
    src = jnp.stack([src[:half], src[half:]], axis=1).reshape(-1)
    dst = jnp.stack([dst[:half], dst[half:]], axis=1).reshape(-1)
    ea_p = jnp.pad(edge_attr, ((0, pad), (0, 0)))
    eps = _edge_proj(ea_p, We, be.reshape(1, _C), lin_e_w,
                     lin_e_b.reshape(_L, 1, _C))
    xp = jnp.pad(x, ((0, _NP - _N), (0, 0)))
    x2 = jnp.stack([xp[:, :_H], xp[:, _H:]])
    for l in range(_L):
        hp2 = _sc_layer(eps[l], src, dst, x2)
        x2 = _node_mlp(hp2, x2, w1[l], b1[l].reshape(1, _C), w2[l],
                       b2[l].reshape(1, _C), ln_g[l].reshape(1, _C),
                       ln_b[l].reshape(1, _C))
    return jnp.concatenate([x2[0, :_N], x2[1, :_N]], axis=1)


# speedup vs baseline: 1.5545x; 1.3441x over previous
import functools

import jax
import jax.numpy as jnp
from jax import lax
from jax.experimental import pallas as pl
from jax.experimental.pallas import tpu as pltpu
from jax.experimental.pallas import tpu_sc as plsc

_N = 10000
_E = 320000
_C = 128
_H = 64
_DE = 16
_L = 3

_TILES = 16
_K = 64
_EPT = 20480
_EP = _EPT * _TILES
_NP = 10112
_RPT = _NP // _TILES


def _edge_proj(ea_p, We, be2, lew, leb):
    Eb = 640
    nblk = _EP // 2 // Eb
    last_valid = _E // Eb - 1

    def body(ea_a, ea_b, we_ref, be_ref, lw_ref, lb_ref, o0, o1, o2):
        ta = jnp.dot(ea_a[...], we_ref[...],
                     preferred_element_type=jnp.float32) + be_ref[...]
        ta = ta * jax.nn.sigmoid(ta)
        tb = jnp.dot(ea_b[...], we_ref[...],
                     preferred_element_type=jnp.float32) + be_ref[...]
        tb = tb * jax.nn.sigmoid(tb)
        outs = (o0, o1, o2)
        for l in range(_L):
            ra = jnp.dot(ta, lw_ref[l],
                         preferred_element_type=jnp.float32) + lb_ref[l]
            rb = jnp.dot(tb, lw_ref[l],
                         preferred_element_type=jnp.float32) + lb_ref[l]
            outs[l][0] = jnp.concatenate([ra[:, :_H], rb[:, :_H]], axis=1)
            outs[l][1] = jnp.concatenate([ra[:, _H:], rb[:, _H:]], axis=1)

    return pl.pallas_call(
        body,
        grid=(nblk,),
        in_specs=[
            pl.BlockSpec((Eb, _DE), lambda i: (i, 0)),
            pl.BlockSpec((Eb, _DE),
                         lambda i: (jnp.minimum(i + nblk, last_valid), 0)),
            pl.BlockSpec((_DE, _C), lambda i: (0, 0)),
            pl.BlockSpec((1, _C), lambda i: (0, 0)),
            pl.BlockSpec((_L, _C, _C), lambda i: (0, 0, 0)),
            pl.BlockSpec((_L, 1, _C), lambda i: (0, 0, 0)),
        ],
        out_specs=[pl.BlockSpec((2, Eb, _C), lambda i: (0, i, 0))] * _L,
        out_shape=[jax.ShapeDtypeStruct((2, _EP // 2, _C), jnp.float32)] * _L,
    )(ea_p, ea_p, We, be2, lew, leb)


def _sc_layer(ep, src, dst, x):
    mesh = plsc.VectorSubcoreMesh(core_axis_name="c", subcore_axis_name="s",
                                  num_cores=2, num_subcores=_TILES)

    @functools.partial(
        pl.kernel,
        out_type=jax.ShapeDtypeStruct((2, _NP, _H), jnp.float32),
        mesh=mesh,
        scratch_types=[
            pltpu.VMEM_SHARED((_NP, _H), jnp.float32),
            pltpu.VMEM_SHARED((_NP, _H), jnp.float32),
            pltpu.VMEM((2, _K // 2, _C), jnp.float32),
            pltpu.VMEM((4, _K, _H), jnp.float32),
            pltpu.VMEM((8, _K // 2), jnp.int32),
            pltpu.VMEM((8, _K // 2), jnp.int32),
            pltpu.VMEM((8, _K // 2), jnp.int32),
            pltpu.VMEM((8, _K // 2), jnp.int32),
            pltpu.SemaphoreType.DMA((2,)),
            pltpu.SemaphoreType.DMA((8,)),
            pltpu.SemaphoreType.DMA((4,)),
            pltpu.SemaphoreType.DMA((4,)),
        ],
    )
    def k(ep_h, src_h, dst_h, x_h, out_h, xsh, agg, epb, xgb, srcba, srcbb,
          dstba, dstbb, sem_ep, sem_ix, sem_g, sem_sc):
        c = lax.axis_index("c")
        s = lax.axis_index("s")
        r0 = s * _RPT
        nck = _EPT // _K
        e_base = s * _EPT

        pltpu.sync_copy(x_h.at[c, pl.ds(r0, _RPT)], xsh.at[pl.ds(r0, _RPT)])
        pltpu.sync_copy(x_h.at[c, pl.ds(r0, _RPT)], agg.at[pl.ds(r0, _RPT)])
        plsc.subcore_barrier()

        def fetch_idx(g, b8):
            p0 = s * (_EPT // 2) + g * (_K // 2)
            q0 = _EP // 2 + p0
            pltpu.async_copy(src_h.at[pl.ds(p0, _K // 2)], srcba.at[b8],
                             sem_ix.at[b8])
            pltpu.async_copy(src_h.at[pl.ds(q0, _K // 2)], srcbb.at[b8],
                             sem_ix.at[b8])
            pltpu.async_copy(dst_h.at[pl.ds(p0, _K // 2)], dstba.at[b8],
                             sem_ix.at[b8])
            pltpu.async_copy(dst_h.at[pl.ds(q0, _K // 2)], dstbb.at[b8],
                             sem_ix.at[b8])

        def fetch_ep(g, b2):
            p0 = s * (_EPT // 2) + g * (_K // 2)
            pltpu.async_copy(ep_h.at[c, pl.ds(p0, _K // 2)], epb.at[b2],
                             sem_ep.at[b2])

        def wait_idx(b8):
            for buf in (srcba, srcbb, dstba, dstbb):
                pltpu.make_async_copy(src_h.at[pl.ds(0, _K // 2)],
                                      buf.at[b8], sem_ix.at[b8]).wait()

        def gather(b4, b8):
            pltpu.async_copy(xsh.at[srcba.at[b8]],
                             xgb.at[b4, pl.ds(0, _K // 2)], sem_g.at[b4])
            pltpu.async_copy(xsh.at[srcbb.at[b8]],
                             xgb.at[b4, pl.ds(_K // 2, _K // 2)],
                             sem_g.at[b4])

        def wait_gather(b4, b8):
            for h in range(2):
                pltpu.make_async_copy(xsh.at[srcba.at[b8]],
                                      xgb.at[b4, pl.ds(h * (_K // 2),
                                                       _K // 2)],
                                      sem_g.at[b4]).wait()

        def wait_ep(b2):
            pltpu.make_async_copy(ep_h.at[0, pl.ds(0, _K // 2)], epb.at[b2],
                                  sem_ep.at[b2]).wait()

        def scatter(b4, b8):
            pltpu.async_copy(xgb.at[b4, pl.ds(0, _K // 2)],
                             agg.at[dstba.at[b8]], sem_sc.at[b4], add=True)
            pltpu.async_copy(xgb.at[b4, pl.ds(_K // 2, _K // 2)],
                             agg.at[dstbb.at[b8]], sem_sc.at[b4], add=True)

        def wait_scatter(b4, b8):
            for h in range(2):
                pltpu.make_async_copy(xgb.at[b4, pl.ds(h * (_K // 2),
                                                       _K // 2)],
                                      agg.at[dstba.at[b8]],
                                      sem_sc.at[b4]).wait()

        def compute(b4, b2):
            @pl.loop(0, _K // 2, unroll=2)
            def _row(r2):
                for h in range(2):
                    for cb in range(_H // 16):
                        sl = pl.ds(cb * 16, 16)
                        esl = pl.ds(h * _H + cb * 16, 16)
                        r = r2 + h * (_K // 2)
                        xgb[b4, r, sl] = jnp.maximum(
                            xgb[b4, r, sl] + epb[b2, r2, esl], 0.0)

        for g in range(4):
            fetch_idx(g, g % 8)
        fetch_ep(0, 0)
        fetch_ep(1, 1)
        wait_idx(0)
        gather(0, 0)
        wait_idx(1)
        gather(1, 1)

        @pl.loop(0, nck // 8)
        def _outer(g8):
            for u in range(8):
                b4 = u % 4
                b2 = u % 2
                g = g8 * 8 + u

                @pl.when(g + 2 < nck)
                def _():
                    wait_idx((u + 2) % 8)

                    @pl.when(g >= 2)
                    def _():
                        wait_scatter((u + 2) % 4, (u + 2) % 8)

                    gather((u + 2) % 4, (u + 2) % 8)

                wait_ep(b2)
                wait_gather(b4, u % 8)
                compute(b4, b2)
                scatter(b4, u % 8)

                @pl.when(g + 2 < nck)
                def _():
                    fetch_ep(g + 2, b2)

                @pl.when(g + 4 < nck)
                def _():
                    fetch_idx(g + 4, (u + 4) % 8)

        for g in range(nck - 4, nck):
            wait_scatter(g % 4, g % 8)
        plsc.subcore_barrier()
        pltpu.sync_copy(agg.at[pl.ds(r0, _RPT)], out_h.at[c, pl.ds(r0, _RPT)])

    return k(ep, src, dst, x)


def _node_mlp(hp, x, w1l, b1l, w2l, b2l, gl, btl):
    Nb = _RPT

    def body(hp_ref, x_ref, w1_ref, b1_ref, w2_ref, b2_ref, g_ref, bt_ref, o_ref):
        xv = jnp.concatenate([x_ref[0], x_ref[1]], axis=-1)
        hp_v = jnp.concatenate([hp_ref[0], hp_ref[1]], axis=-1)
        t = jnp.dot(hp_v, w1_ref[...],
                    preferred_element_type=jnp.float32) + b1_ref[...]
        t = t * jax.nn.sigmoid(t)
        h = jnp.dot(t, w2_ref[...],
                    preferred_element_type=jnp.float32) + b2_ref[...]
        y = xv + h
        mu = jnp.mean(y, axis=-1, keepdims=True)
        d = y - mu
        var = jnp.mean(d * d, axis=-1, keepdims=True)
        o = d * lax.rsqrt(var + 1e-5) * g_ref[...] + bt_ref[...]
        o_ref[0] = o[:, :_H]
        o_ref[1] = o[:, _H:]

    full = lambda i: (0, 0)
    return pl.pallas_call(
        body,
        grid=(_NP // Nb,),
        in_specs=[
            pl.BlockSpec((2, Nb, _H), lambda i: (0, i, 0)),
            pl.BlockSpec((2, Nb, _H), lambda i: (0, i, 0)),
            pl.BlockSpec((_C, _C), full),
            pl.BlockSpec((1, _C), full),
            pl.BlockSpec((_C, _C), full),
            pl.BlockSpec((1, _C), full),
            pl.BlockSpec((1, _C), full),
            pl.BlockSpec((1, _C), full),
        ],
        out_specs=pl.BlockSpec((2, Nb, _H), lambda i: (0, i, 0)),
        out_shape=jax.ShapeDtypeStruct((2, _NP, _H), jnp.float32),
    )(hp, x, w1l, b1l, w2l, b2l, gl, btl)


def kernel(x, edge_index, edge_attr, We, be, lin_e_w, lin_e_b, w1, b1, w2, b2,
           ln_g, ln_b):
    pad = _EP - _E
    src = jnp.pad(edge_index[0], (0, pad))
    pad_dst = _N + (jnp.arange(pad, dtype=jnp.int32) % (_NP - _N))
    dst = jnp.concatenate([edge_index[1], pad_dst])
    eps = _edge_proj(edge_attr, We, be.reshape(1, _C), lin_e_w,
                     lin_e_b.reshape(_L, 1, _C))
    xp = jnp.pad(x, ((0, _NP - _N), (0, 0)))
    x2 = jnp.stack([xp[:, :_H], xp[:, _H:]])
    for l in range(_L):
        hp2 = _sc_layer(eps[l], src, dst, x2)
        x2 = _node_mlp(hp2, x2, w1[l], b1[l].reshape(1, _C), w2[l],
                       b2[l].reshape(1, _C), ln_g[l].reshape(1, _C),
                       ln_b[l].reshape(1, _C))
    return jnp.concatenate([x2[0, :_N], x2[1, :_N]], axis=1)
